# baseline (speedup 1.0000x reference)
import functools

import jax
import jax.numpy as jnp
from jax import lax
from jax.experimental import pallas as pl
from jax.experimental.pallas import tpu as pltpu
from jax.experimental.pallas import tpu_sc as plsc

NW = 32
CH = 128


def _sc_gather_sum(ps, pr, senders2d, receivers2d):
    h = ps.shape[1]
    nch_pad = senders2d.shape[0]
    per_w = nch_pad // NW
    e_pad = nch_pad * CH

    mesh = plsc.VectorSubcoreMesh(core_axis_name="c", subcore_axis_name="s")

    @functools.partial(
        pl.kernel,
        out_type=jax.ShapeDtypeStruct((e_pad, h), jnp.float32),
        mesh=mesh,
        scratch_types=[
            pltpu.VMEM((per_w, CH), jnp.int32),
            pltpu.VMEM((per_w, CH), jnp.int32),
            pltpu.VMEM((CH, h), jnp.float32),
            pltpu.VMEM((CH, h), jnp.float32),
            pltpu.VMEM((CH, h), jnp.float32),
            pltpu.VMEM((CH, h), jnp.float32),
            pltpu.VMEM((CH, h), jnp.float32),
            pltpu.VMEM((CH, h), jnp.float32),
            pltpu.SemaphoreType.DMA,
            pltpu.SemaphoreType.DMA,
            pltpu.SemaphoreType.DMA,
            pltpu.SemaphoreType.DMA,
        ],
    )
    def gather_kernel(ps_hbm, pr_hbm, s2d_hbm, r2d_hbm, z_hbm,
                      sidx, ridx, s0, s1, r0, r1, z0, z1,
                      sg0, sg1, sw0, sw1):
        wid = lax.axis_index("s") * 2 + lax.axis_index("c")
        base = wid * per_w
        srows = (s0, s1)
        rrows = (r0, r1)
        zrows = (z0, z1)
        sem_g = (sg0, sg1)
        sem_w = (sw0, sw1)

        pltpu.sync_copy(s2d_hbm.at[pl.ds(base, per_w), :], sidx)
        pltpu.sync_copy(r2d_hbm.at[pl.ds(base, per_w), :], ridx)

        for b in range(2):
            pltpu.async_copy(ps_hbm.at[sidx.at[b]], srows[b], sem_g[b])
            pltpu.async_copy(pr_hbm.at[ridx.at[b]], rrows[b], sem_g[b])

        @pl.loop(0, per_w, step=2)
        def _(it0):
            for b in range(2):
                it = it0 + b
                pltpu.make_async_copy(ps_hbm.at[pl.ds(0, CH), :], srows[b], sem_g[b]).wait()
                pltpu.make_async_copy(pr_hbm.at[pl.ds(0, CH), :], rrows[b], sem_g[b]).wait()

                @pl.loop(0, CH)
                def _(row):
                    for j in range(h // 16):
                        sl = pl.ds(j * 16, 16)
                        zrows[b][row, sl] = srows[b][row, sl] + rrows[b][row, sl]

                @pl.when(it + 2 < per_w)
                def _():
                    pltpu.async_copy(ps_hbm.at[sidx.at[it + 2]], srows[b], sem_g[b])
                    pltpu.async_copy(pr_hbm.at[ridx.at[it + 2]], rrows[b], sem_g[b])

                @pl.when(it >= 2)
                def _():
                    pltpu.make_async_copy(z_hbm.at[pl.ds(0, CH), :], zrows[b], sem_w[b]).wait()
                pltpu.async_copy(zrows[b], z_hbm.at[pl.ds((base + it) * CH, CH), :], sem_w[b])

        for b in range(2):
            pltpu.make_async_copy(z_hbm.at[pl.ds(0, CH), :], zrows[b], sem_w[b]).wait()

    return gather_kernel(ps, pr, senders2d, receivers2d)


def _sc_scatter(edge_msg, receivers2d, zeros_nh):
    e, h = edge_msg.shape
    n_pad = zeros_nh.shape[0]
    c = 256
    k = c // 128
    n_chunks = e // c
    iters = (n_chunks + NW - 1) // NW
    rows_per_tile = n_pad // 16

    mesh = plsc.VectorSubcoreMesh(core_axis_name="c", subcore_axis_name="s")

    @functools.partial(
        pl.kernel,
        out_type=jax.ShapeDtypeStruct((2, n_pad, h), jnp.float32),
        mesh=mesh,
        scratch_types=[
            pltpu.VMEM((k, 128), jnp.int32),
            pltpu.VMEM((c, h), jnp.float32),
            pltpu.VMEM_SHARED((n_pad, h), jnp.float32),
        ],
    )
    def scatter_kernel(msg_hbm, r2d_hbm, z_hbm, out_hbm, ridx, rows, agg):
        cid = lax.axis_index("c")
        sid = lax.axis_index("s")
        wid = sid * 2 + cid
        pltpu.sync_copy(
            z_hbm.at[pl.ds(sid * rows_per_tile, rows_per_tile), :],
            agg.at[pl.ds(sid * rows_per_tile, rows_per_tile), :],
        )
        plsc.subcore_barrier()

        @pl.loop(0, iters)
        def _(it):
            chunk = it * NW + wid

            @pl.when(chunk < n_chunks)
            def _():
                base = chunk * c
                pltpu.sync_copy(msg_hbm.at[pl.ds(base, c), :], rows)
                pltpu.sync_copy(r2d_hbm.at[pl.ds(chunk * k, k), :], ridx)
                for j in range(k):
                    pltpu.sync_copy(rows.at[pl.ds(j * 128, 128), :], agg.at[ridx.at[j]], add=True)

        plsc.subcore_barrier()
        pltpu.sync_copy(
            agg.at[pl.ds(sid * rows_per_tile, rows_per_tile), :],
            out_hbm.at[cid, pl.ds(sid * rows_per_tile, rows_per_tile), :],
        )

    return scatter_kernel(edge_msg, receivers2d, zeros_nh)


def _tc_preproject(x, w0):
    n, h = x.shape
    t = 1000
    grid = (n // t,)

    def body(x_ref, w0_ref, ps_ref, pr_ref):
        x_v = x_ref[...]
        ps_ref[...] = jnp.dot(x_v, w0_ref[0:h, :], preferred_element_type=jnp.float32)
        pr_ref[...] = jnp.dot(x_v, w0_ref[h:2 * h, :], preferred_element_type=jnp.float32)

    tile = pl.BlockSpec((t, h), lambda i: (i, 0))
    full = pl.BlockSpec((3 * h, h), lambda i: (0, 0))
    return pl.pallas_call(
        body,
        grid=grid,
        in_specs=[tile, full],
        out_specs=[tile, tile],
        out_shape=[
            jax.ShapeDtypeStruct((n, h), jnp.float32),
            jax.ShapeDtypeStruct((n, h), jnp.float32),
        ],
    )(x, w0)


def _tc_edge_mlp(z, ea, w0e, b0, w1, b1, w2, b2, w3, b3):
    e, h = ea.shape
    t = 1280
    grid = (e // t,)

    def body(z_ref, ea_ref, w0e_ref, b0_ref, w1_ref, b1_ref, w2_ref,
             b2_ref, w3_ref, b3_ref, en_ref, eo_ref):
        ea_v = ea_ref[...]
        acc = (
            z_ref[...]
            + jnp.dot(ea_v, w0e_ref[...], preferred_element_type=jnp.float32)
            + b0_ref[...]
        )
        acc = jnp.maximum(acc, 0.0)
        acc = jnp.maximum(jnp.dot(acc, w1_ref[...], preferred_element_type=jnp.float32) + b1_ref[...], 0.0)
        acc = jnp.maximum(jnp.dot(acc, w2_ref[...], preferred_element_type=jnp.float32) + b2_ref[...], 0.0)
        en = jnp.dot(acc, w3_ref[...], preferred_element_type=jnp.float32) + b3_ref[...]
        en_ref[...] = en
        eo_ref[...] = ea_v + en

    full = lambda shape: pl.BlockSpec(shape, lambda i: (0,) * len(shape))
    tile = pl.BlockSpec((t, h), lambda i: (i, 0))
    return pl.pallas_call(
        body,
        grid=grid,
        in_specs=[
            tile, tile,
            full((h, h)), full((1, h)),
            full((h, h)), full((1, h)),
            full((h, h)), full((1, h)),
            full((h, h)), full((1, h)),
        ],
        out_specs=[tile, tile],
        out_shape=[
            jax.ShapeDtypeStruct((e, h), jnp.float32),
            jax.ShapeDtypeStruct((e, h), jnp.float32),
        ],
    )(z, ea, w0e, b0.reshape(1, h), w1, b1.reshape(1, h),
      w2, b2.reshape(1, h), w3, b3.reshape(1, h))


def _tc_node_mlp(x, p0, p1, w0, b0, w1, b1, w2, b2, w3, b3):
    n, h = x.shape
    t = 1000
    grid = (n // t,)

    def body(x_ref, p0_ref, p1_ref, w0_ref, b0_ref, w1_ref, b1_ref, w2_ref,
             b2_ref, w3_ref, b3_ref, xo_ref):
        x_v = x_ref[...]
        agg = p0_ref[...] + p1_ref[...]
        acc = (
            jnp.dot(x_v, w0_ref[0:h, :], preferred_element_type=jnp.float32)
            + jnp.dot(agg, w0_ref[h:2 * h, :], preferred_element_type=jnp.float32)
            + b0_ref[...]
        )
        acc = jnp.maximum(acc, 0.0)
        acc = jnp.maximum(jnp.dot(acc, w1_ref[...], preferred_element_type=jnp.float32) + b1_ref[...], 0.0)
        acc = jnp.maximum(jnp.dot(acc, w2_ref[...], preferred_element_type=jnp.float32) + b2_ref[...], 0.0)
        xo_ref[...] = x_v + jnp.dot(acc, w3_ref[...], preferred_element_type=jnp.float32) + b3_ref[...]

    full = lambda shape: pl.BlockSpec(shape, lambda i: (0,) * len(shape))
    tile = pl.BlockSpec((t, h), lambda i: (i, 0))
    return pl.pallas_call(
        body,
        grid=grid,
        in_specs=[
            tile, tile, tile,
            full((2 * h, h)), full((1, h)),
            full((h, h)), full((1, h)),
            full((h, h)), full((1, h)),
            full((h, h)), full((1, h)),
        ],
        out_specs=tile,
        out_shape=jax.ShapeDtypeStruct((n, h), jnp.float32),
    )(x, p0, p1, w0, b0.reshape(1, h), w1, b1.reshape(1, h),
      w2, b2.reshape(1, h), w3, b3.reshape(1, h))


def kernel(node_attr, edge_index, edge_attr,
           eb_W0, eb_b0, eb_W1, eb_b1, eb_W2, eb_b2, eb_W3, eb_b3,
           nb_W0, nb_b0, nb_W1, nb_b1, nb_W2, nb_b2, nb_W3, nb_b3):
    n, h = node_attr.shape
    e = edge_attr.shape[0]

    nch = e // CH
    nch_pad = ((nch + NW * 8 - 1) // (NW * 8)) * (NW * 8)
    pad = nch_pad * CH - e
    senders2d = jnp.concatenate(
        [edge_index[0], jnp.zeros((pad,), jnp.int32)]).reshape(nch_pad, CH)
    receivers2d = jnp.concatenate(
        [edge_index[1], jnp.zeros((pad,), jnp.int32)]).reshape(nch_pad, CH)

    ps, pr = _tc_preproject(node_attr, eb_W0)
    z_pad = _sc_gather_sum(ps, pr, senders2d, receivers2d)
    en, edge_out = _tc_edge_mlp(z_pad[:e], edge_attr,
                                eb_W0[2 * h:], eb_b0, eb_W1, eb_b1,
                                eb_W2, eb_b2, eb_W3, eb_b3)
    n_pad = ((n + 127) // 128) * 128
    zeros = jnp.zeros((n_pad, h), jnp.float32)
    partials = _sc_scatter(en, receivers2d, zeros)
    x_out = _tc_node_mlp(node_attr, partials[0, :n], partials[1, :n],
                         nb_W0, nb_b0, nb_W1, nb_b1, nb_W2, nb_b2, nb_W3, nb_b3)
    return (x_out, edge_out)
